# baseline (speedup 1.0000x reference)
import jax
import jax.numpy as jnp
from jax import lax
from jax.experimental import pallas as pl
from jax.experimental.pallas import tpu as pltpu
from jax.experimental.pallas import tpu_sc as plsc

B = 16384
D_MOVIE = 64
G = 8
D_GENRE = 16
D_COLL = 16
D_OV = 768
HID = 64
RANK = 64

NC = 2
NS = 16
NW = NC * NS
S = B // NW
C = 128
NCH = S // C
OVR = 8
NBUF = 8
LOOK = 6
NOV = S // OVR


def _sc_big_body(x_hbm, ov_hbm,
                 ov_out,
                 idx_v, ovb_v,
                 sem_i, sem_o):
    wid = lax.axis_index("s") * NC + lax.axis_index("c")
    base = wid * S

    pltpu.sync_copy(x_hbm.at[wid], idx_v)

    def ov_gather(j):
        return pltpu.async_copy(
            ov_hbm.at[idx_v.at[pl.ds(j * OVR, OVR)]], ovb_v.at[j % NBUF],
            sem_i.at[j % NBUF])

    cpi = [None] * NBUF
    cpo = [None] * NBUF
    for j in range(LOOK):
        cpi[j % NBUF] = ov_gather(j)
    for j in range(NOV):
        b = j % NBUF
        cpi[b].wait()
        cpo[b] = pltpu.async_copy(
            ovb_v.at[b], ov_out.at[pl.ds(base + j * OVR, OVR)],
            sem_o.at[b])
        t = j + LOOK
        if t < NOV:
            bt = t % NBUF
            if cpo[bt] is not None:
                cpo[bt].wait()
            cpi[bt] = ov_gather(t)
    for b in range(NBUF):
        if cpo[b] is not None:
            cpo[b].wait()


def _sc_small_body(x_hbm, gpair_hbm, cmap_hbm, movie_hbm, gtab_hbm, ctab_hbm,
                   comb_out,
                   idx_v, idxp_v, meta_v, gtab_v, ccol_v, gacc_v, crow_v,
                   mrow_v, zpad_v,
                   sem_mt0, sem_mt1, sem_cm, sem_mv, sem_tab, sem_out):
    wid = lax.axis_index("s") * NC + lax.axis_index("c")
    base = wid * S

    pltpu.sync_copy(x_hbm.at[wid], idx_v)
    pltpu.sync_copy(gtab_hbm, gtab_v)
    for i in range(S // 16):
        idxp_v[pl.ds(i * 16, 16)] = idx_v[pl.ds(i * 16, 16)] >> 1

    finals = []
    cps_mv = [pltpu.async_copy(movie_hbm.at[idx_v.at[pl.ds(c * C, C)]],
                               mrow_v.at[pl.ds(c * C, C)], sem_mv)
              for c in range(NCH)]
    cps_cm = [pltpu.async_copy(cmap_hbm.at[idx_v.at[pl.ds(c * C, C)]],
                               ccol_v.at[pl.ds(c * C, C)], sem_cm)
              for c in range(NCH)]

    sem_mt = [sem_mt0, sem_mt1]

    def issue_meta(c):
        return pltpu.async_copy(gpair_hbm.at[idxp_v.at[pl.ds(c * C, C)]],
                                meta_v.at[c % 2], sem_mt[c % 2])

    zero16 = jnp.zeros((16,), jnp.float32)

    def z_body(r, _):
        zpad_v[r, pl.ds(0, 16)] = zero16
        zpad_v[r, pl.ds(16, 16)] = zero16
        return 0

    lax.fori_loop(0, C, z_body, 0, unroll=4)

    pend_meta = issue_meta(0)
    cp_ctab = [None] * NCH
    for c in range(NCH):
        nxt_meta = issue_meta(c + 1) if c + 1 < NCH else None
        pend_meta.wait()
        pend_meta = nxt_meta
        mv = meta_v.at[c % 2]
        if c == 0:
            for cp in cps_cm:
                cp.wait()
            cp_ctab = [pltpu.async_copy(
                ctab_hbm.at[ccol_v.at[pl.ds(i * C, C)]], crow_v.at[i],
                sem_tab) for i in range(NCH)]

        def gen_body(g, _):
            pv = idx_v[pl.ds(c * C + g * 16, 16)] & 1
            for rr in range(16):
                v = mv[g * 16 + rr, :]
                acc_a = gtab_v[v[0], :]
                acc_b = gtab_v[v[G], :]
                for k in range(1, G):
                    acc_a = acc_a + gtab_v[v[k], :]
                    acc_b = acc_b + gtab_v[v[G + k], :]
                gacc_v[c, g * 16 + rr, :] = jnp.where(pv[rr] > 0, acc_b,
                                                      acc_a)
            return 0

        lax.fori_loop(0, C // 16, gen_body, 0)
        rows = pl.ds(base + c * C, C)
        finals.append(pltpu.async_copy(
            gacc_v.at[c], comb_out.at[rows, pl.ds(0, D_GENRE)], sem_out))
        finals.append(pltpu.async_copy(
            zpad_v, comb_out.at[rows, pl.ds(96, 32)], sem_out))

    for c in range(NCH):
        cp_ctab[c].wait()
        finals.append(pltpu.async_copy(
            crow_v.at[c],
            comb_out.at[pl.ds(base + c * C, C), pl.ds(D_GENRE, D_COLL)],
            sem_out))
    for cp in cps_mv:
        cp.wait()
    finals.append(pltpu.async_copy(
        mrow_v, comb_out.at[pl.ds(base, S), pl.ds(32, D_MOVIE)], sem_out))
    for cp in finals:
        cp.wait()


@jax.jit
def _sc_gather(x, movie_table, gmap, cmap, ov_emb, genre_table, coll_table):
    x2 = x.reshape(NW, S)
    mesh = plsc.VectorSubcoreMesh(core_axis_name="c", subcore_axis_name="s")
    big = pl.kernel(
        _sc_big_body,
        out_type=[
            jax.ShapeDtypeStruct((B, D_OV), jnp.float32),
        ],
        mesh=mesh,
        scratch_types=[
            pltpu.VMEM((S,), jnp.int32),
            pltpu.VMEM((NBUF, OVR, D_OV), jnp.float32),
            pltpu.SemaphoreType.DMA((NBUF,)),
            pltpu.SemaphoreType.DMA((NBUF,)),
        ],
    )
    small = pl.kernel(
        _sc_small_body,
        out_type=[
            jax.ShapeDtypeStruct((B, 128), jnp.float32),
        ],
        mesh=mesh,
        compiler_params=pltpu.CompilerParams(use_tc_tiling_on_sc=False),
        scratch_types=[
            pltpu.VMEM((S,), jnp.int32),
            pltpu.VMEM((S,), jnp.int32),
            pltpu.VMEM((2, C, 16), jnp.int32),
            pltpu.VMEM((20, D_GENRE), jnp.float32),
            pltpu.VMEM((S,), jnp.int32),
            pltpu.VMEM((NCH, C, D_GENRE), jnp.float32),
            pltpu.VMEM((NCH, C, D_COLL), jnp.float32),
            pltpu.VMEM((S, D_MOVIE), jnp.float32),
            pltpu.VMEM((C, 32), jnp.float32),
            pltpu.SemaphoreType.DMA,
            pltpu.SemaphoreType.DMA,
            pltpu.SemaphoreType.DMA,
            pltpu.SemaphoreType.DMA,
            pltpu.SemaphoreType.DMA,
            pltpu.SemaphoreType.DMA,
        ],
    )
    (ov,) = big(x2, ov_emb)
    gpair = gmap.reshape(gmap.shape[0] // 2, 2 * G)
    (comb,) = small(x2, gpair, cmap, movie_table, genre_table, coll_table)
    return comb, ov


def _mlp_body(cb_ref, ov_ref, w1cb_ref, w1o_ref, b1_ref, w2_ref, b2_ref,
              out_ref):
    h = jnp.dot(ov_ref[...], w1o_ref[...], preferred_element_type=jnp.float32)
    h = h + jnp.dot(cb_ref[...], w1cb_ref[...],
                    preferred_element_type=jnp.float32)
    h = jnp.maximum(h + b1_ref[...], 0.0)
    out_ref[...] = jnp.dot(h, w2_ref[...],
                           preferred_element_type=jnp.float32) + b2_ref[...]


TB = 2048


@jax.jit
def _mlp(cb, ov, w1cb, w1o, b1, w2, b2):
    grid = (B // TB,)
    bspec = lambda d: pl.BlockSpec((TB, d), lambda i: (i, 0))
    wspec = lambda r, c: pl.BlockSpec((r, c), lambda i: (0, 0))
    return pl.pallas_call(
        _mlp_body,
        grid=grid,
        in_specs=[
            bspec(128), bspec(D_OV),
            wspec(128, HID), wspec(D_OV, HID), wspec(1, HID),
            wspec(HID, RANK), wspec(1, RANK),
        ],
        out_specs=pl.BlockSpec((TB, RANK), lambda i: (i, 0)),
        out_shape=jax.ShapeDtypeStruct((B, RANK), jnp.float32),
    )(cb, ov, w1cb, w1o, b1, w2, b2)


def kernel(x, movie_table, genres_map, collection_map, ov_emb,
           genre_table, coll_table, W1, b1, W2, b2):
    comb, ov = _sc_gather(x, movie_table, genres_map, collection_map,
                          ov_emb, genre_table, coll_table)
    w1g = W1[D_MOVIE:D_MOVIE + D_GENRE] * (1.0 / G)
    w1c = W1[D_MOVIE + D_GENRE:D_MOVIE + D_GENRE + D_COLL]
    w1m = W1[:D_MOVIE]
    w1o = W1[D_MOVIE + D_GENRE + D_COLL:]
    w1cb = jnp.concatenate(
        [w1g, w1c, w1m, jnp.zeros((32, HID), jnp.float32)], axis=0)
    return _mlp(comb, ov, w1cb, w1o,
                b1.reshape(1, HID), W2, b2.reshape(1, RANK))
